# baseline (speedup 1.0000x reference)
import functools

import jax
import jax.numpy as jnp
from jax import lax
from jax.experimental import pallas as pl
from jax.experimental.pallas import tpu as pltpu
from jax.experimental.pallas import tpu_sc as plsc

N = 50000
E = 800000
G = 128
D = 256
LAYERS = 5
VOCAB = 5000
SEQ = 5
NUM_TYPES = 98
MAX_DEPTH = 20

NC = 2
NS = 16
NW = NC * NS

NP = 53248
BR = 256
NBLK = NP // BR
BK = 128
VECS = 1568
EPT = VECS * 16
EPAD = NW * EPT
NO = 64
OFFW = 96
CAP = EPT + (NO + 1) * BK + 512
WN = 512
CB = 1664
DT = 1024
GK = 64
PADROW = N

RB = 512
NB = NP // RB
PB = 1000
VB = VOCAB

_sc_mesh = plsc.VectorSubcoreMesh(core_axis_name="c", subcore_axis_name="s")
_sc_params = pltpu.CompilerParams(needs_layout_passes=False)


@functools.partial(
    pl.kernel,
    mesh=_sc_mesh,
    out_type=(
        jax.ShapeDtypeStruct((NW * CAP,), jnp.int32),
        jax.ShapeDtypeStruct((NW * CAP,), jnp.int32),
        jax.ShapeDtypeStruct((NW * OFFW,), jnp.int32),
    ),
    scratch_types=(
        pltpu.VMEM((EPT,), jnp.int32),
        pltpu.VMEM((EPT,), jnp.int32),
        pltpu.VMEM((CAP,), jnp.int32),
        pltpu.VMEM((CAP,), jnp.int32),
        pltpu.VMEM((OFFW,), jnp.int32),
    ),
    compiler_params=_sc_params,
)
def _bucket_kernel(esrc_hbm, edst_hbm, bsrc_hbm, bdst_hbm, boffs_hbm,
                   src_v, dst_v, bsrc_v, bdst_v, offs_v):
    cid = lax.axis_index("c")
    sid = lax.axis_index("s")
    wid = sid * NC + cid
    base = wid * EPT
    pltpu.sync_copy(esrc_hbm.at[pl.ds(base, EPT)], src_v)
    pltpu.sync_copy(edst_hbm.at[pl.ds(base, EPT)], dst_v)

    lanes = lax.iota(jnp.int32, 16)
    zero16 = jnp.zeros((16,), jnp.int32)
    dead16 = jnp.full((16,), 1 << 20, jnp.int32)
    offs_vecs = [jnp.zeros((16,), jnp.int32) for _ in range(OFFW // 16)]
    cur = jnp.int32(0)
    for c in range(NO):
        def body(i, cur):
            d = dst_v[pl.ds(i * 16, 16)]
            s_ = src_v[pl.ds(i * 16, 16)]
            m = jnp.bitwise_and(lax.shift_right_logical(d, 8), NO - 1) == c
            mi = m.astype(jnp.int32)
            incl = plsc.cumsum(mi)
            idx = cur + incl - mi
            plsc.store_scatter(bsrc_v, [idx], s_, mask=m)
            plsc.store_scatter(bdst_v, [idx], d, mask=m)
            return cur + incl[15]
        cur = lax.fori_loop(0, VECS, body, cur)
        bound = jnp.bitwise_and(cur + (BK - 1), -BK)
        npad = (bound - cur + 15) >> 4
        def pbody(p, cur):
            bsrc_v[pl.ds(cur, 16)] = zero16
            bdst_v[pl.ds(cur, 16)] = dead16
            return cur + 16
        lax.fori_loop(0, npad, pbody, cur)
        cur = bound
        vi, li = (c + 1) // 16, (c + 1) % 16
        offs_vecs[vi] = jnp.where(lanes == li, bound, offs_vecs[vi])

    def sbody(p, cur):
        bsrc_v[pl.ds(cur, 16)] = zero16
        bdst_v[pl.ds(cur, 16)] = dead16
        return cur + 16
    lax.fori_loop(0, (CAP - cur) >> 4, sbody, cur)

    for vi in range(OFFW // 16):
        offs_v[pl.ds(vi * 16, 16)] = offs_vecs[vi]
    obase = wid * CAP
    pltpu.sync_copy(bsrc_v, bsrc_hbm.at[pl.ds(obase, CAP)])
    pltpu.sync_copy(bdst_v, bdst_hbm.at[pl.ds(obase, CAP)])
    pltpu.sync_copy(offs_v, boffs_hbm.at[pl.ds(wid * OFFW, OFFW)])


@functools.partial(
    pl.kernel,
    mesh=_sc_mesh,
    out_type=jax.ShapeDtypeStruct((NP, D), jnp.float32),
    scratch_types=(
        pltpu.VMEM((BR, D), jnp.float32),
        pltpu.VMEM((2, GK, D), jnp.float32),
        pltpu.VMEM((2 * WN,), jnp.int32),
        pltpu.VMEM((2 * WN,), jnp.int32),
        pltpu.VMEM((CB,), jnp.int32),
        pltpu.VMEM((CB,), jnp.int32),
        pltpu.VMEM((NW * OFFW,), jnp.int32),
        pltpu.SemaphoreType.DMA((2,)),
        pltpu.SemaphoreType.DMA((2,)),
    ),
    compiler_params=_sc_params,
)
def _agg_kernel(h_hbm, bsrc_hbm, bdst_hbm, boffs_hbm, agg_hbm,
                acc_v, rows_v, lsrc_v, ldst_v, csrc_v, cldst_v,
                offs_v, lsem, gsem):
    cid = lax.axis_index("c")
    sid = lax.axis_index("s")
    wid = sid * NC + cid
    pltpu.sync_copy(boffs_hbm, offs_v)
    padrow16 = jnp.full((16,), 0, jnp.int32) + (PADROW + jnp.bitwise_and(wid, 15))
    zero16 = jnp.zeros((16,), jnp.int32)
    lanes = lax.iota(jnp.int32, 16)

    def gather_copy(fb, start):
        fbase = pl.multiple_of(fb << 6, 8)
        par = jnp.bitwise_and(fb, 1)
        mk = pltpu.async_copy if start else pltpu.make_async_copy
        return mk(h_hbm.at[csrc_v.at[pl.ds(fbase, GK)]],
                  rows_v.at[par], gsem.at[par])

    def drain(ccur):
        nfb = ccur >> 6
        @pl.when(nfb > 0)
        def _():
            gather_copy(0, True)
        def dblk(fb, _):
            par = jnp.bitwise_and(fb, 1)
            @pl.when(fb + 1 < nfb)
            def _():
                gather_copy(fb + 1, True)
            gather_copy(fb, False).wait()
            fbase = fb << 6
            def agrp(grp, _):
                ldv = cldst_v[pl.ds(fbase + grp * 16, 16)]
                for lane in range(16):
                    ld = ldv[lane]
                    e = grp * 16 + lane
                    for j in range(D // 16):
                        plsc.addupdate(acc_v.at[ld, pl.ds(j * 16, 16)],
                                       rows_v[par, e, pl.ds(j * 16, 16)])
                return 0
            lax.fori_loop(0, GK // 16, agrp, 0)
            return 0
        lax.fori_loop(0, nfb, dblk, 0)
        tail = ccur - (nfb << 6)
        tbase = nfb << 6
        @pl.when(nfb > 0)
        def _():
            for tg in range(4):
                @pl.when(tg * 16 < tail)
                def _():
                    csrc_v[pl.ds(tg * 16, 16)] = csrc_v[pl.ds(tbase + tg * 16, 16)]
                    cldst_v[pl.ds(tg * 16, 16)] = cldst_v[pl.ds(tbase + tg * 16, 16)]
        return tail

    def per_block(bi, _):
        g = wid + bi * NW
        @pl.when(g < NBLK)
        def _():
            zf = jnp.zeros((16,), jnp.float32)
            def zbody(i, _):
                acc_v[i >> 4, pl.ds(jnp.bitwise_and(i, 15) << 4, 16)] = zf
                return 0
            lax.fori_loop(0, BR * (D // 16), zbody, 0)

            slot = wid + ((jnp.bitwise_and(bi, 1)) << 5)
            def per_writer(w, ccur):
                wl = w * OFFW + slot
                start = jnp.max(plsc.load_gather(offs_v, [zero16 + wl]))
                end = jnp.max(plsc.load_gather(offs_v, [zero16 + wl + 1]))
                wbase = w * CAP + start
                nb = (end - start + WN - 1) >> 9

                def win_copies(b, start):
                    par = jnp.bitwise_and(b, 1)
                    pos = pl.multiple_of(wbase + b * WN, 8)
                    mk = pltpu.async_copy if start else pltpu.make_async_copy
                    cs = mk(bsrc_hbm.at[pl.ds(pos, WN)],
                            lsrc_v.at[pl.ds(pl.multiple_of(par * WN, 8), WN)],
                            lsem.at[par])
                    cd = mk(bdst_hbm.at[pl.ds(pos, WN)],
                            ldst_v.at[pl.ds(pl.multiple_of(par * WN, 8), WN)],
                            lsem.at[par])
                    return cs, cd

                @pl.when(nb > 0)
                def _():
                    win_copies(0, True)

                def per_win(b, ccur):
                    par = jnp.bitwise_and(b, 1)
                    @pl.when(b + 1 < nb)
                    def _():
                        win_copies(b + 1, True)
                    cs, cd = win_copies(b, False)
                    cs.wait()
                    cd.wait()
                    for grp in range(WN // 16):
                        d = ldst_v[pl.ds(par * WN + grp * 16, 16)]
                        s_ = lsrc_v[pl.ds(par * WN + grp * 16, 16)]
                        m = lax.shift_right_logical(d, 8) == g
                        mi = m.astype(jnp.int32)
                        incl = plsc.cumsum(mi)
                        idx = ccur + incl - mi
                        plsc.store_scatter(csrc_v, [idx], s_, mask=m)
                        plsc.store_scatter(cldst_v, [idx],
                                           jnp.bitwise_and(d, BR - 1), mask=m)
                        ccur = ccur + incl[15]
                    return lax.cond(ccur >= DT, drain, lambda c: c, ccur)

                return lax.fori_loop(0, nb, per_win, ccur)

            ccur = lax.fori_loop(0, NW, per_writer, jnp.int32(0))
            @pl.when(ccur > 0)
            def _():
                for pg in range(4):
                    csrc_v[pl.ds(ccur + pg * 16, 16)] = padrow16
                    cldst_v[pl.ds(ccur + pg * 16, 16)] = zero16
                drain(jnp.bitwise_and(ccur + GK - 1, -GK))
            pltpu.sync_copy(
                acc_v, agg_hbm.at[pl.ds(pl.multiple_of(g * BR, 8), BR)])
        return 0

    lax.fori_loop(0, 7, per_block, 0)


def _embed_body(nt_ref, nd_ref, tt_ref, dt_ref, o_ref):
    t = nt_ref[0, 0, :]
    dep = nd_ref[0, 0, :]
    iota = lax.broadcasted_iota(jnp.int32, (RB, 128), 1)
    oh_t = (t[:, None] == iota).astype(jnp.float32)
    oh_d = (dep[:, None] == iota).astype(jnp.float32)
    o_ref[...] = (jnp.dot(oh_t, tt_ref[...], preferred_element_type=jnp.float32)
                  + jnp.dot(oh_d, dt_ref[...], preferred_element_type=jnp.float32))


def _mlp_body(h_ref, a_ref, w1_ref, b1_ref, w2_ref, b2_ref, eps_ref, o_ref, *, last):
    z = h_ref[...] * (1.0 + eps_ref[0, 0]) + a_ref[...]
    z1 = jnp.maximum(
        jnp.dot(z, w1_ref[...], preferred_element_type=jnp.float32) + b1_ref[...], 0.0)
    out = jnp.dot(z1, w2_ref[...], preferred_element_type=jnp.float32) + b2_ref[...]
    if not last:
        out = jnp.maximum(out, 0.0)
    rid = pl.program_id(0) * RB + lax.broadcasted_iota(jnp.int32, (RB, 1), 0)
    o_ref[...] = jnp.where(rid < N, out, 0.0)


def _pool_body(g_ref, h_ref, o_ref, sums, cnts):
    i = pl.program_id(0)
    @pl.when(i == 0)
    def _():
        sums[...] = jnp.zeros_like(sums)
        cnts[...] = jnp.zeros_like(cnts)
    g = g_ref[0, 0, :]
    iota = lax.broadcasted_iota(jnp.int32, (PB, G), 1)
    oh = (g[:, None] == iota).astype(jnp.float32)
    sums[...] += lax.dot_general(oh, h_ref[...], (((0,), (0,)), ((), ())),
                                 preferred_element_type=jnp.float32)
    cnts[...] += lax.dot_general(oh, jnp.ones((PB, 1), jnp.float32),
                                 (((0,), (0,)), ((), ())),
                                 preferred_element_type=jnp.float32)
    @pl.when(i == pl.num_programs(0) - 1)
    def _():
        o_ref[...] = sums[...] / jnp.maximum(cnts[...], 1.0)


def _head_body(rep_ref, wp_ref, bp_ref, o_ref):
    o_ref[0] = (jnp.dot(rep_ref[...], wp_ref[0], preferred_element_type=jnp.float32)
                + bp_ref[0])


def kernel(node_type, node_depth, edge_index, graph_ids,
           type_table, depth_table, W1, b1, W2, b2, eps, Wp, bp):
    f32 = jnp.float32
    esrc = jnp.pad(edge_index[0], (0, EPAD - E))
    edst = jnp.pad(edge_index[1], (0, EPAD - E), constant_values=NP)
    nt3 = jnp.pad(node_type, (0, NP - N), constant_values=127).reshape(NB, 1, RB)
    nd3 = jnp.pad(node_depth, (0, NP - N), constant_values=127).reshape(NB, 1, RB)
    tt_pad = jnp.pad(type_table, ((0, 128 - NUM_TYPES), (0, 0)))
    dt_pad = jnp.pad(depth_table, ((0, 128 - MAX_DEPTH), (0, 0)))
    gid3 = graph_ids.reshape(N // PB, 1, PB)

    bsrc, bdst, boffs = _bucket_kernel(esrc, edst)

    h = pl.pallas_call(
        _embed_body,
        grid=(NB,),
        in_specs=[
            pl.BlockSpec((1, 1, RB), lambda i: (i, 0, 0)),
            pl.BlockSpec((1, 1, RB), lambda i: (i, 0, 0)),
            pl.BlockSpec((128, D), lambda i: (0, 0)),
            pl.BlockSpec((128, D), lambda i: (0, 0)),
        ],
        out_specs=pl.BlockSpec((RB, D), lambda i: (i, 0)),
        out_shape=jax.ShapeDtypeStruct((NP, D), f32),
    )(nt3, nd3, tt_pad, dt_pad)

    for l in range(LAYERS):
        agg = _agg_kernel(h, bsrc, bdst, boffs)
        h = pl.pallas_call(
            functools.partial(_mlp_body, last=(l == LAYERS - 1)),
            grid=(NB,),
            in_specs=[
                pl.BlockSpec((RB, D), lambda i: (i, 0)),
                pl.BlockSpec((RB, D), lambda i: (i, 0)),
                pl.BlockSpec((D, 2 * D), lambda i: (0, 0)),
                pl.BlockSpec((1, 2 * D), lambda i: (0, 0)),
                pl.BlockSpec((2 * D, D), lambda i: (0, 0)),
                pl.BlockSpec((1, D), lambda i: (0, 0)),
                pl.BlockSpec((1, 1), lambda i: (0, 0)),
            ],
            out_specs=pl.BlockSpec((RB, D), lambda i: (i, 0)),
            out_shape=jax.ShapeDtypeStruct((NP, D), f32),
        )(h, agg, W1[l], b1[l].reshape(1, 2 * D), W2[l], b2[l].reshape(1, D),
          eps[l].reshape(1, 1))

    rep = pl.pallas_call(
        _pool_body,
        grid=(N // PB,),
        in_specs=[
            pl.BlockSpec((1, 1, PB), lambda i: (i, 0, 0)),
            pl.BlockSpec((PB, D), lambda i: (i, 0)),
        ],
        out_specs=pl.BlockSpec((G, D), lambda i: (0, 0)),
        out_shape=jax.ShapeDtypeStruct((G, D), f32),
        scratch_shapes=[pltpu.VMEM((G, D), f32), pltpu.VMEM((G, 1), f32)],
    )(gid3, h)

    preds = pl.pallas_call(
        _head_body,
        grid=(SEQ,),
        in_specs=[
            pl.BlockSpec((G, D), lambda s: (0, 0)),
            pl.BlockSpec((1, D, VB), lambda s: (s, 0, 0)),
            pl.BlockSpec((1, 1, VB), lambda s: (s, 0, 0)),
        ],
        out_specs=pl.BlockSpec((1, G, VB), lambda s: (s, 0, 0)),
        out_shape=jax.ShapeDtypeStruct((SEQ, G, VOCAB), f32),
    )(rep, Wp, bp.reshape(SEQ, 1, VOCAB))
    return preds


# speedup vs baseline: 1.0211x; 1.0211x over previous
import functools

import jax
import jax.numpy as jnp
from jax import lax
from jax.experimental import pallas as pl
from jax.experimental.pallas import tpu as pltpu
from jax.experimental.pallas import tpu_sc as plsc

N = 50000
E = 800000
G = 128
D = 256
LAYERS = 5
VOCAB = 5000
SEQ = 5
NUM_TYPES = 98
MAX_DEPTH = 20

NC = 2
NS = 16
NW = NC * NS

NP = 53248
BR = 256
NBLK = NP // BR
BK = 128
VECS = 1568
EPT = VECS * 16
EPAD = NW * EPT
NO = 32
OFFW = 48
CAP = EPT + (NO + 1) * BK + 512
WN = 512
CB = 1664
DT = 1024
GK = 64
PADROW = N

RB = 512
NB = NP // RB
PB = 1000
VB = VOCAB

_sc_mesh = plsc.VectorSubcoreMesh(core_axis_name="c", subcore_axis_name="s")
_sc_params = pltpu.CompilerParams(needs_layout_passes=False)


@functools.partial(
    pl.kernel,
    mesh=_sc_mesh,
    out_type=(
        jax.ShapeDtypeStruct((NW * CAP,), jnp.int32),
        jax.ShapeDtypeStruct((NW * CAP,), jnp.int32),
        jax.ShapeDtypeStruct((NW * OFFW,), jnp.int32),
    ),
    scratch_types=(
        pltpu.VMEM((EPT,), jnp.int32),
        pltpu.VMEM((EPT,), jnp.int32),
        pltpu.VMEM((CAP,), jnp.int32),
        pltpu.VMEM((CAP,), jnp.int32),
        pltpu.VMEM((OFFW,), jnp.int32),
    ),
    compiler_params=_sc_params,
)
def _bucket_kernel(esrc_hbm, edst_hbm, bsrc_hbm, bdst_hbm, boffs_hbm,
                   src_v, dst_v, bsrc_v, bdst_v, offs_v):
    cid = lax.axis_index("c")
    sid = lax.axis_index("s")
    wid = sid * NC + cid
    base = wid * EPT
    pltpu.sync_copy(esrc_hbm.at[pl.ds(base, EPT)], src_v)
    pltpu.sync_copy(edst_hbm.at[pl.ds(base, EPT)], dst_v)

    lanes = lax.iota(jnp.int32, 16)
    zero16 = jnp.zeros((16,), jnp.int32)
    dead16 = jnp.full((16,), 1 << 20, jnp.int32)
    offs_vecs = [jnp.zeros((16,), jnp.int32) for _ in range(OFFW // 16)]
    cur = jnp.int32(0)
    for c in range(NO):
        def body(i, cur):
            d = dst_v[pl.ds(i * 16, 16)]
            s_ = src_v[pl.ds(i * 16, 16)]
            m = jnp.bitwise_and(lax.shift_right_logical(d, 8), NO - 1) == c
            mi = m.astype(jnp.int32)
            incl = plsc.cumsum(mi)
            idx = cur + incl - mi
            plsc.store_scatter(bsrc_v, [idx], s_, mask=m)
            plsc.store_scatter(bdst_v, [idx], d, mask=m)
            return cur + incl[15]
        cur = lax.fori_loop(0, VECS, body, cur)
        bound = jnp.bitwise_and(cur + (BK - 1), -BK)
        npad = (bound - cur + 15) >> 4
        def pbody(p, cur):
            bsrc_v[pl.ds(cur, 16)] = zero16
            bdst_v[pl.ds(cur, 16)] = dead16
            return cur + 16
        lax.fori_loop(0, npad, pbody, cur)
        cur = bound
        vi, li = (c + 1) // 16, (c + 1) % 16
        offs_vecs[vi] = jnp.where(lanes == li, bound, offs_vecs[vi])

    def sbody(p, cur):
        bsrc_v[pl.ds(cur, 16)] = zero16
        bdst_v[pl.ds(cur, 16)] = dead16
        return cur + 16
    lax.fori_loop(0, (CAP - cur) >> 4, sbody, cur)

    for vi in range(OFFW // 16):
        offs_v[pl.ds(vi * 16, 16)] = offs_vecs[vi]
    obase = wid * CAP
    pltpu.sync_copy(bsrc_v, bsrc_hbm.at[pl.ds(obase, CAP)])
    pltpu.sync_copy(bdst_v, bdst_hbm.at[pl.ds(obase, CAP)])
    pltpu.sync_copy(offs_v, boffs_hbm.at[pl.ds(wid * OFFW, OFFW)])


@functools.partial(
    pl.kernel,
    mesh=_sc_mesh,
    out_type=jax.ShapeDtypeStruct((NP, D), jnp.float32),
    scratch_types=(
        pltpu.VMEM((BR, D), jnp.float32),
        pltpu.VMEM((2, GK, D), jnp.float32),
        pltpu.VMEM((2 * WN,), jnp.int32),
        pltpu.VMEM((2 * WN,), jnp.int32),
        pltpu.VMEM((CB,), jnp.int32),
        pltpu.VMEM((CB,), jnp.int32),
        pltpu.VMEM((NW * OFFW,), jnp.int32),
        pltpu.SemaphoreType.DMA((2,)),
        pltpu.SemaphoreType.DMA((2,)),
    ),
    compiler_params=_sc_params,
)
def _agg_kernel(h_hbm, bsrc_hbm, bdst_hbm, boffs_hbm, agg_hbm,
                acc_v, rows_v, lsrc_v, ldst_v, csrc_v, cldst_v,
                offs_v, lsem, gsem):
    cid = lax.axis_index("c")
    sid = lax.axis_index("s")
    wid = sid * NC + cid
    pltpu.sync_copy(boffs_hbm, offs_v)
    padrow16 = jnp.full((16,), 0, jnp.int32) + (PADROW + jnp.bitwise_and(wid, 15))
    zero16 = jnp.zeros((16,), jnp.int32)
    lanes = lax.iota(jnp.int32, 16)

    def gather_copy(fb, start):
        fbase = pl.multiple_of(fb << 6, 8)
        par = jnp.bitwise_and(fb, 1)
        mk = pltpu.async_copy if start else pltpu.make_async_copy
        return mk(h_hbm.at[csrc_v.at[pl.ds(fbase, GK)]],
                  rows_v.at[par], gsem.at[par])

    def drain(ccur):
        nfb = ccur >> 6
        @pl.when(nfb > 0)
        def _():
            gather_copy(0, True)
        def dblk(fb, _):
            par = jnp.bitwise_and(fb, 1)
            @pl.when(fb + 1 < nfb)
            def _():
                gather_copy(fb + 1, True)
            gather_copy(fb, False).wait()
            fbase = fb << 6
            def agrp(grp, _):
                ldv = cldst_v[pl.ds(fbase + grp * 16, 16)]
                for lane in range(16):
                    ld = ldv[lane]
                    e = grp * 16 + lane
                    for j in range(D // 16):
                        plsc.addupdate(acc_v.at[ld, pl.ds(j * 16, 16)],
                                       rows_v[par, e, pl.ds(j * 16, 16)])
                return 0
            lax.fori_loop(0, GK // 16, agrp, 0)
            return 0
        lax.fori_loop(0, nfb, dblk, 0)
        tail = ccur - (nfb << 6)
        tbase = nfb << 6
        @pl.when(nfb > 0)
        def _():
            for tg in range(4):
                @pl.when(tg * 16 < tail)
                def _():
                    csrc_v[pl.ds(tg * 16, 16)] = csrc_v[pl.ds(tbase + tg * 16, 16)]
                    cldst_v[pl.ds(tg * 16, 16)] = cldst_v[pl.ds(tbase + tg * 16, 16)]
        return tail

    def per_block(bi, _):
        g = wid + bi * NW
        @pl.when(g < NBLK)
        def _():
            zf = jnp.zeros((16,), jnp.float32)
            def zbody(i, _):
                acc_v[i >> 4, pl.ds(jnp.bitwise_and(i, 15) << 4, 16)] = zf
                return 0
            lax.fori_loop(0, BR * (D // 16), zbody, 0)

            def per_writer(w, ccur):
                wl = w * OFFW + wid
                start = jnp.max(plsc.load_gather(offs_v, [zero16 + wl]))
                end = jnp.max(plsc.load_gather(offs_v, [zero16 + wl + 1]))
                wbase = w * CAP + start
                nb = (end - start + WN - 1) >> 9

                def win_copies(b, start):
                    par = jnp.bitwise_and(b, 1)
                    pos = pl.multiple_of(wbase + b * WN, 8)
                    mk = pltpu.async_copy if start else pltpu.make_async_copy
                    cs = mk(bsrc_hbm.at[pl.ds(pos, WN)],
                            lsrc_v.at[pl.ds(pl.multiple_of(par * WN, 8), WN)],
                            lsem.at[par])
                    cd = mk(bdst_hbm.at[pl.ds(pos, WN)],
                            ldst_v.at[pl.ds(pl.multiple_of(par * WN, 8), WN)],
                            lsem.at[par])
                    return cs, cd

                @pl.when(nb > 0)
                def _():
                    win_copies(0, True)

                def per_win(b, ccur):
                    par = jnp.bitwise_and(b, 1)
                    @pl.when(b + 1 < nb)
                    def _():
                        win_copies(b + 1, True)
                    cs, cd = win_copies(b, False)
                    cs.wait()
                    cd.wait()
                    for grp in range(WN // 16):
                        d = ldst_v[pl.ds(par * WN + grp * 16, 16)]
                        s_ = lsrc_v[pl.ds(par * WN + grp * 16, 16)]
                        m = lax.shift_right_logical(d, 8) == g
                        mi = m.astype(jnp.int32)
                        incl = plsc.cumsum(mi)
                        idx = ccur + incl - mi
                        plsc.store_scatter(csrc_v, [idx], s_, mask=m)
                        plsc.store_scatter(cldst_v, [idx],
                                           jnp.bitwise_and(d, BR - 1), mask=m)
                        ccur = ccur + incl[15]
                    return lax.cond(ccur >= DT, drain, lambda c: c, ccur)

                return lax.fori_loop(0, nb, per_win, ccur)

            ccur = lax.fori_loop(0, NW, per_writer, jnp.int32(0))
            @pl.when(ccur > 0)
            def _():
                for pg in range(4):
                    csrc_v[pl.ds(ccur + pg * 16, 16)] = padrow16
                    cldst_v[pl.ds(ccur + pg * 16, 16)] = zero16
                drain(jnp.bitwise_and(ccur + GK - 1, -GK))
            pltpu.sync_copy(
                acc_v, agg_hbm.at[pl.ds(pl.multiple_of(g * BR, 8), BR)])
        return 0

    lax.fori_loop(0, 7, per_block, 0)


def _embed_body(nt_ref, nd_ref, tt_ref, dt_ref, o_ref):
    t = nt_ref[0, 0, :]
    dep = nd_ref[0, 0, :]
    iota = lax.broadcasted_iota(jnp.int32, (RB, 128), 1)
    oh_t = (t[:, None] == iota).astype(jnp.float32)
    oh_d = (dep[:, None] == iota).astype(jnp.float32)
    o_ref[...] = (jnp.dot(oh_t, tt_ref[...], preferred_element_type=jnp.float32)
                  + jnp.dot(oh_d, dt_ref[...], preferred_element_type=jnp.float32))


def _mlp_body(h_ref, a_ref, w1_ref, b1_ref, w2_ref, b2_ref, eps_ref, o_ref, *, last):
    z = h_ref[...] * (1.0 + eps_ref[0, 0]) + a_ref[...]
    z1 = jnp.maximum(
        jnp.dot(z, w1_ref[...], preferred_element_type=jnp.float32) + b1_ref[...], 0.0)
    out = jnp.dot(z1, w2_ref[...], preferred_element_type=jnp.float32) + b2_ref[...]
    if not last:
        out = jnp.maximum(out, 0.0)
    rid = pl.program_id(0) * RB + lax.broadcasted_iota(jnp.int32, (RB, 1), 0)
    o_ref[...] = jnp.where(rid < N, out, 0.0)


def _pool_body(g_ref, h_ref, o_ref, sums, cnts):
    i = pl.program_id(0)
    @pl.when(i == 0)
    def _():
        sums[...] = jnp.zeros_like(sums)
        cnts[...] = jnp.zeros_like(cnts)
    g = g_ref[0, 0, :]
    iota = lax.broadcasted_iota(jnp.int32, (PB, G), 1)
    oh = (g[:, None] == iota).astype(jnp.float32)
    sums[...] += lax.dot_general(oh, h_ref[...], (((0,), (0,)), ((), ())),
                                 preferred_element_type=jnp.float32)
    cnts[...] += lax.dot_general(oh, jnp.ones((PB, 1), jnp.float32),
                                 (((0,), (0,)), ((), ())),
                                 preferred_element_type=jnp.float32)
    @pl.when(i == pl.num_programs(0) - 1)
    def _():
        o_ref[...] = sums[...] / jnp.maximum(cnts[...], 1.0)


def _head_body(rep_ref, wp_ref, bp_ref, o_ref):
    o_ref[0] = (jnp.dot(rep_ref[...], wp_ref[0], preferred_element_type=jnp.float32)
                + bp_ref[0])


def kernel(node_type, node_depth, edge_index, graph_ids,
           type_table, depth_table, W1, b1, W2, b2, eps, Wp, bp):
    f32 = jnp.float32
    esrc = jnp.pad(edge_index[0], (0, EPAD - E))
    edst = jnp.pad(edge_index[1], (0, EPAD - E), constant_values=NP)
    nt3 = jnp.pad(node_type, (0, NP - N), constant_values=127).reshape(NB, 1, RB)
    nd3 = jnp.pad(node_depth, (0, NP - N), constant_values=127).reshape(NB, 1, RB)
    tt_pad = jnp.pad(type_table, ((0, 128 - NUM_TYPES), (0, 0)))
    dt_pad = jnp.pad(depth_table, ((0, 128 - MAX_DEPTH), (0, 0)))
    gid3 = graph_ids.reshape(N // PB, 1, PB)

    bsrc, bdst, boffs = _bucket_kernel(esrc, edst)

    h = pl.pallas_call(
        _embed_body,
        grid=(NB,),
        in_specs=[
            pl.BlockSpec((1, 1, RB), lambda i: (i, 0, 0)),
            pl.BlockSpec((1, 1, RB), lambda i: (i, 0, 0)),
            pl.BlockSpec((128, D), lambda i: (0, 0)),
            pl.BlockSpec((128, D), lambda i: (0, 0)),
        ],
        out_specs=pl.BlockSpec((RB, D), lambda i: (i, 0)),
        out_shape=jax.ShapeDtypeStruct((NP, D), f32),
    )(nt3, nd3, tt_pad, dt_pad)

    for l in range(LAYERS):
        agg = _agg_kernel(h, bsrc, bdst, boffs)
        h = pl.pallas_call(
            functools.partial(_mlp_body, last=(l == LAYERS - 1)),
            grid=(NB,),
            in_specs=[
                pl.BlockSpec((RB, D), lambda i: (i, 0)),
                pl.BlockSpec((RB, D), lambda i: (i, 0)),
                pl.BlockSpec((D, 2 * D), lambda i: (0, 0)),
                pl.BlockSpec((1, 2 * D), lambda i: (0, 0)),
                pl.BlockSpec((2 * D, D), lambda i: (0, 0)),
                pl.BlockSpec((1, D), lambda i: (0, 0)),
                pl.BlockSpec((1, 1), lambda i: (0, 0)),
            ],
            out_specs=pl.BlockSpec((RB, D), lambda i: (i, 0)),
            out_shape=jax.ShapeDtypeStruct((NP, D), f32),
        )(h, agg, W1[l], b1[l].reshape(1, 2 * D), W2[l], b2[l].reshape(1, D),
          eps[l].reshape(1, 1))

    rep = pl.pallas_call(
        _pool_body,
        grid=(N // PB,),
        in_specs=[
            pl.BlockSpec((1, 1, PB), lambda i: (i, 0, 0)),
            pl.BlockSpec((PB, D), lambda i: (i, 0)),
        ],
        out_specs=pl.BlockSpec((G, D), lambda i: (0, 0)),
        out_shape=jax.ShapeDtypeStruct((G, D), f32),
        scratch_shapes=[pltpu.VMEM((G, D), f32), pltpu.VMEM((G, 1), f32)],
    )(gid3, h)

    preds = pl.pallas_call(
        _head_body,
        grid=(SEQ,),
        in_specs=[
            pl.BlockSpec((G, D), lambda s: (0, 0)),
            pl.BlockSpec((1, D, VB), lambda s: (s, 0, 0)),
            pl.BlockSpec((1, 1, VB), lambda s: (s, 0, 0)),
        ],
        out_specs=pl.BlockSpec((1, G, VB), lambda s: (s, 0, 0)),
        out_shape=jax.ShapeDtypeStruct((SEQ, G, VOCAB), f32),
    )(rep, Wp, bp.reshape(SEQ, 1, VOCAB))
    return preds
